# Initial kernel scaffold; baseline (speedup 1.0000x reference)
import functools

import jax
import jax.numpy as jnp
from jax import lax
from jax.experimental import pallas as pl
from jax.experimental.pallas import tpu as pltpu
from jax.experimental.pallas import tpu_sc as plsc

NC = 2
NS = 16
CH = 128
DEGW = 16

_MESH = plsc.VectorSubcoreMesh(
    core_axis_name="c", subcore_axis_name="s", num_cores=NC, num_subcores=NS)


def _deg_body(nrow, ept, dst_hbm, deg_hbm, dst_v, ones_v, z_v, deg_sp):
  c = lax.axis_index("c")
  s = lax.axis_index("s")
  stripe = nrow // NS

  @pl.when(c == 0)
  def _init():
    def fill(i, _):
      ones_v[i, :] = jnp.ones((16,), jnp.float32)
      return 0
    lax.fori_loop(0, CH, fill, 0)

    def zfill(i, _):
      z_v[i, :] = jnp.zeros((16,), jnp.float32)
      return 0
    lax.fori_loop(0, stripe, zfill, 0)
    pltpu.sync_copy(z_v, deg_sp.at[pl.ds(s * stripe, stripe), :])

  plsc.subcore_barrier()

  @pl.when(c == 0)
  def _scatter():
    base = s * ept

    def chunk(k, _):
      pltpu.sync_copy(dst_hbm.at[pl.ds(base + k * CH, CH)], dst_v)
      pltpu.sync_copy(ones_v, deg_sp.at[dst_v], add=True)
      return 0
    lax.fori_loop(0, ept // CH, chunk, 0)

  plsc.subcore_barrier()

  @pl.when(c == 0)
  def _dump():
    pltpu.sync_copy(deg_sp.at[pl.ds(s * stripe, stripe), :],
                    deg_hbm.at[pl.ds(s * stripe, stripe), :])


def _agg_body(nrow, ept, src_hbm, dst_hbm, h0_hbm, h1_hbm, a0_hbm, a1_hbm,
              src_v, dst_v, gbuf, z_v, acc_sp, sem):
  c = lax.axis_index("c")
  s = lax.axis_index("s")
  stripe = nrow // NS

  def zfill(i, _):
    for j in range(8):
      z_v[i, pl.ds(j * 16, 16)] = jnp.zeros((16,), jnp.float32)
    return 0
  lax.fori_loop(0, CH, zfill, 0)
  for r in range(stripe // CH):
    pltpu.sync_copy(z_v, acc_sp.at[pl.ds(s * stripe + r * CH, CH), :])

  plsc.subcore_barrier()

  base = s * ept

  def chunk(k, _):
    e0 = base + k * CH
    pltpu.sync_copy(src_hbm.at[pl.ds(e0, CH)], src_v)
    pltpu.sync_copy(dst_hbm.at[pl.ds(e0, CH)], dst_v)

    @pl.when(c == 0)
    def _g0():
      pltpu.async_copy(h0_hbm.at[src_v], gbuf, sem).wait()

    @pl.when(c == 1)
    def _g1():
      pltpu.async_copy(h1_hbm.at[src_v], gbuf, sem).wait()

    pltpu.sync_copy(gbuf, acc_sp.at[dst_v], add=True)
    return 0
  lax.fori_loop(0, ept // CH, chunk, 0)

  plsc.subcore_barrier()

  @pl.when(c == 0)
  def _d0():
    pltpu.sync_copy(acc_sp.at[pl.ds(s * stripe, stripe), :],
                    a0_hbm.at[pl.ds(s * stripe, stripe), :])

  @pl.when(c == 1)
  def _d1():
    pltpu.sync_copy(acc_sp.at[pl.ds(s * stripe, stripe), :],
                    a1_hbm.at[pl.ds(s * stripe, stripe), :])


def _prelude_body(x_ref, m_ref, g_ref, bt_ref, w_ref, deg_ref, h0_ref, h1_ref):
  xb = x_ref[...]
  mu = jnp.mean(xb, axis=1, keepdims=True)
  xc = xb - mu
  var = jnp.mean(xc * xc, axis=1, keepdims=True)
  y = xc * lax.rsqrt(var + 1e-5) * g_ref[...] + bt_ref[...]
  y = jnp.maximum(y, 0.0) * m_ref[...]
  h = jnp.dot(y, w_ref[...], preferred_element_type=jnp.float32)
  dinv = lax.rsqrt(deg_ref[...][:, 0:1] + 1.0)
  hs = h * dinv
  half = hs.shape[1] // 2
  h0_ref[...] = hs[:, :half]
  h1_ref[...] = hs[:, half:]


def _epi_body(a0_ref, a1_ref, h0_ref, h1_ref, deg_ref, b_ref, o_ref):
  dinv = lax.rsqrt(deg_ref[...][:, 0:1] + 1.0)
  o0 = dinv * (a0_ref[...] + h0_ref[...])
  o1 = dinv * (a1_ref[...] + h1_ref[...])
  o_ref[...] = jnp.concatenate([o0, o1], axis=1) + b_ref[...]


def kernel(x, edge_index, dropout_mask, gamma, beta, W, b):
  n, d = x.shape
  e = edge_index.shape[1]
  half = d // 2

  nrow = (n // (NS * CH) + 1) * (NS * CH)
  ept = -(-e // (NS * CH)) * CH
  e_pad = NS * ept

  src = edge_index[0].astype(jnp.int32)
  dst = edge_index[1].astype(jnp.int32)
  pad = e_pad - e
  src_p = jnp.concatenate([src, jnp.zeros((pad,), jnp.int32)])
  dst_p = jnp.concatenate([dst, jnp.full((pad,), n, jnp.int32)])

  deg_kernel = pl.kernel(
      functools.partial(_deg_body, nrow, ept),
      out_type=jax.ShapeDtypeStruct((nrow, DEGW), jnp.float32),
      mesh=_MESH,
      scratch_types=[
          pltpu.VMEM((CH,), jnp.int32),
          pltpu.VMEM((CH, DEGW), jnp.float32),
          pltpu.VMEM((nrow // NS, DEGW), jnp.float32),
          pltpu.VMEM_SHARED((nrow, DEGW), jnp.float32),
      ],
  )
  deg16 = deg_kernel(dst_p)

  rb = 400
  grid = n // rb
  g2 = gamma.reshape(1, d)
  bt2 = beta.reshape(1, d)
  h0, h1 = pl.pallas_call(
      _prelude_body,
      grid=(grid,),
      in_specs=[
          pl.BlockSpec((rb, d), lambda i: (i, 0)),
          pl.BlockSpec((rb, d), lambda i: (i, 0)),
          pl.BlockSpec((1, d), lambda i: (0, 0)),
          pl.BlockSpec((1, d), lambda i: (0, 0)),
          pl.BlockSpec((d, d), lambda i: (0, 0)),
          pl.BlockSpec((rb, DEGW), lambda i: (i, 0)),
      ],
      out_specs=[
          pl.BlockSpec((rb, half), lambda i: (i, 0)),
          pl.BlockSpec((rb, half), lambda i: (i, 0)),
      ],
      out_shape=[
          jax.ShapeDtypeStruct((n, half), jnp.float32),
          jax.ShapeDtypeStruct((n, half), jnp.float32),
      ],
  )(x, dropout_mask, g2, bt2, W, deg16)

  agg_kernel = pl.kernel(
      functools.partial(_agg_body, nrow, ept),
      out_type=(
          jax.ShapeDtypeStruct((nrow, half), jnp.float32),
          jax.ShapeDtypeStruct((nrow, half), jnp.float32),
      ),
      mesh=_MESH,
      scratch_types=[
          pltpu.VMEM((CH,), jnp.int32),
          pltpu.VMEM((CH,), jnp.int32),
          pltpu.VMEM((CH, half), jnp.float32),
          pltpu.VMEM((CH, half), jnp.float32),
          pltpu.VMEM_SHARED((nrow, half), jnp.float32),
          pltpu.SemaphoreType.DMA,
      ],
  )
  a0, a1 = agg_kernel(src_p, dst_p, h0, h1)

  b2 = b.reshape(1, d)
  out = pl.pallas_call(
      _epi_body,
      grid=(grid,),
      in_specs=[
          pl.BlockSpec((rb, half), lambda i: (i, 0)),
          pl.BlockSpec((rb, half), lambda i: (i, 0)),
          pl.BlockSpec((rb, half), lambda i: (i, 0)),
          pl.BlockSpec((rb, half), lambda i: (i, 0)),
          pl.BlockSpec((rb, DEGW), lambda i: (i, 0)),
          pl.BlockSpec((1, d), lambda i: (0, 0)),
      ],
      out_specs=pl.BlockSpec((rb, d), lambda i: (i, 0)),
      out_shape=jax.ShapeDtypeStruct((n, d), jnp.float32),
  )(a0, a1, h0, h1, deg16, b2)
  return out


# speedup vs baseline: 3.2112x; 3.2112x over previous
import functools

import jax
import jax.numpy as jnp
from jax import lax
from jax.experimental import pallas as pl
from jax.experimental.pallas import tpu as pltpu
from jax.experimental.pallas import tpu_sc as plsc

NC = 2
NS = 16
CH = 64
DEGW = 16

_MESH = plsc.VectorSubcoreMesh(
    core_axis_name="c", subcore_axis_name="s", num_cores=NC, num_subcores=NS)


def _gather_body(ept_w, h_hbm, idx_hbm, hg_hbm, idx_v, gbuf, sem):
  c = lax.axis_index("c")
  s = lax.axis_index("s")
  wid = s * NC + c
  base = wid * ept_w

  def chunk(k, _):
    e0 = base + k * CH
    pltpu.sync_copy(idx_hbm.at[pl.ds(e0, CH)], idx_v)
    pltpu.async_copy(h_hbm.at[idx_v], gbuf, sem).wait()
    pltpu.sync_copy(gbuf, hg_hbm.at[pl.ds(e0, CH), :])
    return 0
  lax.fori_loop(0, ept_w // CH, chunk, 0)


def _prelude_body(x_ref, m_ref, g_ref, bt_ref, w_ref, deg_ref, h_ref):
  xb = x_ref[...]
  mu = jnp.mean(xb, axis=1, keepdims=True)
  xc = xb - mu
  var = jnp.mean(xc * xc, axis=1, keepdims=True)
  y = xc * lax.rsqrt(var + 1e-5) * g_ref[...] + bt_ref[...]
  y = jnp.maximum(y, 0.0) * m_ref[...]
  h = jnp.dot(y, w_ref[...], preferred_element_type=jnp.float32)
  dinv = lax.rsqrt(deg_ref[...][:, 0:1] + 1.0)
  h_ref[...] = h * dinv


def _epi_body(a_ref, h_ref, deg_ref, b_ref, o_ref):
  dinv = lax.rsqrt(deg_ref[...][:, 0:1] + 1.0)
  o_ref[...] = dinv * (a_ref[...] + h_ref[...]) + b_ref[...]


def kernel(x, edge_index, dropout_mask, gamma, beta, W, b):
  n, d = x.shape
  e = edge_index.shape[1]

  src = edge_index[0].astype(jnp.int32)
  dst = edge_index[1].astype(jnp.int32)

  e_pad = -(-e // (NC * NS * CH)) * (NC * NS * CH)
  src_p = jnp.concatenate([src, jnp.zeros((e_pad - e,), jnp.int32)])

  degv = jax.ops.segment_sum(jnp.ones((e,), jnp.float32), dst,
                             num_segments=n)
  deg16 = jnp.broadcast_to(degv[:, None], (n, DEGW))

  rb = 400
  grid = n // rb
  g2 = gamma.reshape(1, d)
  bt2 = beta.reshape(1, d)
  hs = pl.pallas_call(
      _prelude_body,
      grid=(grid,),
      in_specs=[
          pl.BlockSpec((rb, d), lambda i: (i, 0)),
          pl.BlockSpec((rb, d), lambda i: (i, 0)),
          pl.BlockSpec((1, d), lambda i: (0, 0)),
          pl.BlockSpec((1, d), lambda i: (0, 0)),
          pl.BlockSpec((d, d), lambda i: (0, 0)),
          pl.BlockSpec((rb, DEGW), lambda i: (i, 0)),
      ],
      out_specs=pl.BlockSpec((rb, d), lambda i: (i, 0)),
      out_shape=jax.ShapeDtypeStruct((n, d), jnp.float32),
  )(x, dropout_mask, g2, bt2, W, deg16)

  ept_w = e_pad // (NC * NS)
  gather_kernel = pl.kernel(
      functools.partial(_gather_body, ept_w),
      out_type=jax.ShapeDtypeStruct((e_pad, d), jnp.float32),
      mesh=_MESH,
      scratch_types=[
          pltpu.VMEM((CH,), jnp.int32),
          pltpu.VMEM((CH, d), jnp.float32),
          pltpu.SemaphoreType.DMA,
      ],
  )
  hg = gather_kernel(hs, src_p)

  agg = jax.ops.segment_sum(hg[:e], dst, num_segments=n)

  b2 = b.reshape(1, d)
  out = pl.pallas_call(
      _epi_body,
      grid=(grid,),
      in_specs=[
          pl.BlockSpec((rb, d), lambda i: (i, 0)),
          pl.BlockSpec((rb, d), lambda i: (i, 0)),
          pl.BlockSpec((rb, DEGW), lambda i: (i, 0)),
          pl.BlockSpec((1, d), lambda i: (0, 0)),
      ],
      out_specs=pl.BlockSpec((rb, d), lambda i: (i, 0)),
      out_shape=jax.ShapeDtypeStruct((n, d), jnp.float32),
  )(agg, hs, deg16, b2)
  return out
